# baseline (speedup 1.0000x reference)
import jax
import jax.numpy as jnp
import numpy as np
from jax import lax
from jax.experimental import pallas as pl
from jax.experimental.pallas import tpu as pltpu
from jax.experimental.pallas import tpu_sc as plsc

VMIN = np.float32(-100000.0)
VMAX = np.float32(100000.0)
RANGE = np.float32(200000.0)
NBUCKETS = 100000000
DIV = np.float32(NBUCKETS - 1)
WIN = 64

ROWS = 16384
COLS = 512

_NC = 2
_NS = 16
_NL = 16
_NW = _NC * _NS
_QPW = ROWS // _NW
_VPW = _QPW // _NL

CHUNK = 32
_NCHUNK = _QPW // CHUNK


def _bucket_vals(idx):
    t = idx.astype(jnp.float32) / DIV
    b = VMIN * (np.float32(1.0) - t) + VMAX * t
    return jnp.where(idx == NBUCKETS - 1, VMAX, b)


def _splat_elem(ref, i):
    return plsc.load_gather(ref, [jnp.full((_NL,), i, jnp.int32)])


def _body(z_hbm, out_hbm, q_v, s_v, buf0, buf1, sem0, sem1):
    wid = lax.axis_index("s") * _NC + lax.axis_index("c")
    base0 = wid * _QPW
    pltpu.sync_copy(z_hbm.at[pl.ds(base0, _QPW)], q_v)

    def seeds_body(v, carry):
        q = q_v[pl.ds(v * _NL, _NL)]
        g = (q - VMIN) / RANGE * DIV
        base = jnp.clip(g.astype(jnp.int32) - WIN // 2, 0, NBUCKETS - WIN)
        res = jnp.zeros((_NL,), jnp.int32)
        w = WIN // 2
        while w >= 1:
            b = _bucket_vals(base + (res + (w - 1)))
            res = jnp.where(b < q, res + w, res)
            w //= 2
        b = _bucket_vals(base + res)
        res = jnp.where(b < q, res + 1, res)
        s_v[pl.ds(v * _NL, _NL)] = base + res
        return carry

    lax.fori_loop(0, _VPW, seeds_body, 0)

    bufs = (buf0, buf1)
    sems = (sem0, sem1)
    handles = [None, None]
    for k in range(_NCHUNK):
        buf = bufs[k % 2]
        if handles[k % 2] is not None:
            handles[k % 2].wait()

        @plsc.parallel_loop(0, CHUNK, step=1, unroll=2)
        def fill_body(r, _k=k, _buf=buf):
            val = _splat_elem(s_v, _k * CHUNK + r)
            for cc in range(COLS // _NL):
                _buf[r, pl.ds(cc * _NL, _NL)] = val

        handles[k % 2] = pltpu.async_copy(
            buf, out_hbm.at[pl.ds(base0 + k * CHUNK, CHUNK)], sems[k % 2])

    handles[(_NCHUNK - 2) % 2].wait()
    handles[(_NCHUNK - 1) % 2].wait()


_sc_call = pl.kernel(
    _body,
    mesh=plsc.VectorSubcoreMesh(core_axis_name="c", subcore_axis_name="s"),
    out_type=jax.ShapeDtypeStruct((ROWS, COLS), jnp.int32),
    scratch_types=[
        pltpu.VMEM((_QPW,), jnp.float32),
        pltpu.VMEM((_QPW,), jnp.int32),
        pltpu.VMEM((CHUNK, COLS), jnp.int32),
        pltpu.VMEM((CHUNK, COLS), jnp.int32),
        pltpu.SemaphoreType.DMA,
        pltpu.SemaphoreType.DMA,
    ],
    compiler_params=pltpu.CompilerParams(needs_layout_passes=False),
)


def kernel(z, c):
    del c
    return _sc_call(z[:, 0])


# speedup vs baseline: 1.0512x; 1.0512x over previous
import jax
import jax.numpy as jnp
import numpy as np
from jax import lax
from jax.experimental import pallas as pl
from jax.experimental.pallas import tpu as pltpu
from jax.experimental.pallas import tpu_sc as plsc

VMIN = np.float32(-100000.0)
VMAX = np.float32(100000.0)
RANGE = np.float32(200000.0)
NBUCKETS = 100000000
DIV = np.float32(NBUCKETS - 1)
WIN = 64

ROWS = 16384
COLS = 512

_NC = 2
_NS = 16
_NL = 16
_NW = _NC * _NS
_QPW = ROWS // _NW
_VPW = _QPW // _NL

CHUNK = 64
_NCHUNK = _QPW // CHUNK


def _bucket_vals(idx):
    t = idx.astype(jnp.float32) / DIV
    b = VMIN * (np.float32(1.0) - t) + VMAX * t
    return jnp.where(idx == NBUCKETS - 1, VMAX, b)


def _splat_elem(ref, i):
    return plsc.load_gather(ref, [jnp.full((_NL,), i, jnp.int32)])


def _body(z_hbm, out_hbm, q_v, s_v, buf0, buf1, sem0, sem1):
    wid = lax.axis_index("s") * _NC + lax.axis_index("c")
    base0 = wid * _QPW
    pltpu.sync_copy(z_hbm.at[pl.ds(base0, _QPW)], q_v)

    def seeds_body(v, carry):
        q = q_v[pl.ds(v * _NL, _NL)]
        g = (q - VMIN) / RANGE * DIV
        base = jnp.clip(g.astype(jnp.int32) - WIN // 2, 0, NBUCKETS - WIN)
        res = jnp.zeros((_NL,), jnp.int32)
        w = WIN // 2
        while w >= 1:
            b = _bucket_vals(base + (res + (w - 1)))
            res = jnp.where(b < q, res + w, res)
            w //= 2
        b = _bucket_vals(base + res)
        res = jnp.where(b < q, res + 1, res)
        s_v[pl.ds(v * _NL, _NL)] = base + res
        return carry

    lax.fori_loop(0, _VPW, seeds_body, 0)

    bufs = (buf0, buf1)
    sems = (sem0, sem1)
    handles = [None, None]
    for k in range(_NCHUNK):
        buf = bufs[k % 2]
        if handles[k % 2] is not None:
            handles[k % 2].wait()

        @plsc.parallel_loop(0, CHUNK, step=1, unroll=2)
        def fill_body(r, _k=k, _buf=buf):
            val = _splat_elem(s_v, _k * CHUNK + r)
            for cc in range(COLS // _NL):
                _buf[r, pl.ds(cc * _NL, _NL)] = val

        handles[k % 2] = pltpu.async_copy(
            buf, out_hbm.at[pl.ds(base0 + k * CHUNK, CHUNK)], sems[k % 2])

    handles[(_NCHUNK - 2) % 2].wait()
    handles[(_NCHUNK - 1) % 2].wait()


_sc_call = pl.kernel(
    _body,
    mesh=plsc.VectorSubcoreMesh(core_axis_name="c", subcore_axis_name="s"),
    out_type=jax.ShapeDtypeStruct((ROWS, COLS), jnp.int32),
    scratch_types=[
        pltpu.VMEM((_QPW,), jnp.float32),
        pltpu.VMEM((_QPW,), jnp.int32),
        pltpu.VMEM((CHUNK, COLS), jnp.int32),
        pltpu.VMEM((CHUNK, COLS), jnp.int32),
        pltpu.SemaphoreType.DMA,
        pltpu.SemaphoreType.DMA,
    ],
    compiler_params=pltpu.CompilerParams(needs_layout_passes=False),
)


def kernel(z, c):
    del c
    return _sc_call(z[:, 0])
